# baseline (speedup 1.0000x reference)
import jax
import jax.numpy as jnp
from jax import lax
from jax.experimental import pallas as pl
from jax.experimental.pallas import tpu as pltpu

_W = 64
_SHRINK = {5: 96, 4: 80, 3: 72, 2: 72, 1: 72, 0: 72}


def _roll_up(z, sh):
    return jnp.concatenate([z[sh:], z[:sh]], axis=0)


def _shift_down_to_window(z, bit_masks):
    for b in range(5, -1, -1):
        sh = 1 << b
        z = jnp.where(bit_masks[b], _roll_up(z, sh), z)
        ns = _SHRINK[b]
        if z.shape[0] > ns:
            z = z[:ns]
    return z[:_W]


def _cmpex_large(w, k, j):
    C = w.shape[1]
    if k == _W:
        M = _W // (2 * j)
        wv = w.reshape(M, 2, j, C)
        a, b = wv[:, 0], wv[:, 1]
        mn, mx = jnp.minimum(a, b), jnp.maximum(a, b)
        return jnp.concatenate([mn[:, None], mx[:, None]], axis=1).reshape(_W, C)
    G = _W // (2 * k)
    M = k // (2 * j)
    wv = w.reshape(G, 2, M, 2, j, C)
    a, b = wv[:, :, :, 0], wv[:, :, :, 1]
    mn, mx = jnp.minimum(a, b), jnp.maximum(a, b)
    h0 = jnp.concatenate([mn[:, 0:1], mx[:, 1:2]], axis=1)
    h1 = jnp.concatenate([mx[:, 0:1], mn[:, 1:2]], axis=1)
    return jnp.concatenate(
        [h0[:, :, :, None], h1[:, :, :, None]], axis=3).reshape(_W, C)


def _sort64(w, row):
    k = 2
    while k <= _W:
        j = k // 2
        while j > 0:
            if j >= 8:
                w = _cmpex_large(w, k, j)
                j //= 2
                continue
            up = _roll_up(w, j)
            dn = _roll_up(w, _W - j)
            bitj = (row & j) == 0
            p = jnp.where(bitj, up, dn)
            if k == _W:
                take_min = bitj
            else:
                take_min = ((row & k) == 0) == bitj
            w = jnp.where(take_min, jnp.minimum(w, p), jnp.maximum(w, p))
            j //= 2
        k *= 2
    return w


def _windowed_sort_kernel(v_ref, o_ref, y_ref):
    x = v_ref[0]
    L, C = x.shape
    n_strips = L // _W
    lane = lax.broadcasted_iota(jnp.int32, (1, C), 1) & (_W - 1)
    row = lax.broadcasted_iota(jnp.int32, (_W, 1), 0)
    fwd_masks = [(lane & (1 << b)) != 0 for b in range(6)]
    amt = (_W - lane) & (_W - 1)
    inv_masks = [(amt & (1 << b)) != 0 for b in range(6)]
    o_zero = lane == 0

    def sort_strip(z):
        return _sort64(_shift_down_to_window(z, fwd_masks), row)

    def unshift_strip(z):
        shifted = _shift_down_to_window(z, inv_masks)
        return jnp.where(o_zero, z[_W:], shifted)

    def sort_body(s, _):
        z = v_ref[0, pl.ds(_W * s, 2 * _W), :]
        y_ref[pl.ds(_W * s, _W), :] = sort_strip(z)
        return 0

    lax.fori_loop(0, n_strips - 1, sort_body, 0)
    z_last = jnp.concatenate([x[L - _W:], x[:_W]], axis=0)
    y_ref[L - _W:, :] = sort_strip(z_last)

    def unshift_body(s, _):
        z = y_ref[pl.ds(_W * (s - 1), 2 * _W), :]
        o_ref[0, pl.ds(_W * s, _W), :] = unshift_strip(z)
        return 0

    z0 = jnp.concatenate([y_ref[L - _W:, :], y_ref[: _W, :]], axis=0)
    o_ref[0, : _W, :] = unshift_strip(z0)
    lax.fori_loop(1, n_strips, unshift_body, 0)


def kernel(q, k, v):
    B, L, D = v.shape
    C = 256
    grid = (B, D // C)
    return pl.pallas_call(
        _windowed_sort_kernel,
        grid=grid,
        in_specs=[pl.BlockSpec((1, L, C), lambda b, c: (b, 0, c))],
        out_specs=pl.BlockSpec((1, L, C), lambda b, c: (b, 0, c)),
        out_shape=jax.ShapeDtypeStruct(v.shape, v.dtype),
        scratch_shapes=[pltpu.VMEM((L, C), jnp.float32)],
        compiler_params=pltpu.CompilerParams(
            dimension_semantics=("parallel", "parallel"),
        ),
    )(v)


# speedup vs baseline: 1.0258x; 1.0258x over previous
import jax
import jax.numpy as jnp
from jax import lax
from jax.experimental import pallas as pl
from jax.experimental.pallas import tpu as pltpu

_W = 64
_SHRINK = {5: 96, 4: 80, 3: 72, 2: 72, 1: 72, 0: 72}


def _roll_up(z, sh):
    return jnp.concatenate([z[sh:], z[:sh]], axis=0)


def _shift_down_to_window(z, bit_masks):
    for b in range(5, -1, -1):
        sh = 1 << b
        z = jnp.where(bit_masks[b], _roll_up(z, sh), z)
        ns = _SHRINK[b]
        if z.shape[0] > ns:
            z = z[:ns]
    return z[:_W]


def _sort64(w, row):
    k = 2
    while k <= _W:
        j = k // 2
        while j > 0:
            up = _roll_up(w, j)
            dn = _roll_up(w, _W - j)
            bitj = (row & j) == 0
            p = jnp.where(bitj, up, dn)
            if k == _W:
                take_min = bitj
            else:
                take_min = ((row & k) == 0) == bitj
            w = jnp.where(take_min, jnp.minimum(w, p), jnp.maximum(w, p))
            j //= 2
        k *= 2
    return w


def _windowed_sort_kernel(v_ref, o_ref, y_ref):
    x = v_ref[0]
    L, C = x.shape
    n_strips = L // _W
    lane = lax.broadcasted_iota(jnp.int32, (1, C), 1) & (_W - 1)
    row = lax.broadcasted_iota(jnp.int32, (_W, 1), 0)
    fwd_masks = [(lane & (1 << b)) != 0 for b in range(6)]
    amt = (_W - lane) & (_W - 1)
    inv_masks = [(amt & (1 << b)) != 0 for b in range(6)]
    o_zero = lane == 0

    def sort_strip(z):
        return _sort64(_shift_down_to_window(z, fwd_masks), row)

    def unshift_strip(z):
        shifted = _shift_down_to_window(z, inv_masks)
        return jnp.where(o_zero, z[_W:], shifted)

    y_ref[: _W, :] = sort_strip(v_ref[0, : 2 * _W, :])

    def body(s, _):
        z = v_ref[0, pl.ds(_W * s, 2 * _W), :]
        y_ref[pl.ds(_W * s, _W), :] = sort_strip(z)
        z2 = y_ref[pl.ds(_W * (s - 1), 2 * _W), :]
        o_ref[0, pl.ds(_W * s, _W), :] = unshift_strip(z2)
        return 0

    lax.fori_loop(1, n_strips - 1, body, 0)
    z_last = jnp.concatenate([x[L - _W:], x[:_W]], axis=0)
    y_ref[L - _W:, :] = sort_strip(z_last)
    o_ref[0, L - _W:, :] = unshift_strip(y_ref[L - 2 * _W:, :])
    z0 = jnp.concatenate([y_ref[L - _W:, :], y_ref[: _W, :]], axis=0)
    o_ref[0, : _W, :] = unshift_strip(z0)


def kernel(q, k, v):
    B, L, D = v.shape
    C = 256
    grid = (B, D // C)
    return pl.pallas_call(
        _windowed_sort_kernel,
        grid=grid,
        in_specs=[pl.BlockSpec((1, L, C), lambda b, c: (b, 0, c))],
        out_specs=pl.BlockSpec((1, L, C), lambda b, c: (b, 0, c)),
        out_shape=jax.ShapeDtypeStruct(v.shape, v.dtype),
        scratch_shapes=[pltpu.VMEM((L, C), jnp.float32)],
        compiler_params=pltpu.CompilerParams(
            dimension_semantics=("parallel", "parallel"),
        ),
    )(v)
